# Initial kernel scaffold; baseline (speedup 1.0000x reference)
import functools

import jax
import jax.numpy as jnp
from jax import lax
from jax.experimental import pallas as pl
from jax.experimental.pallas import tpu as pltpu
from jax.experimental.pallas import tpu_sc as plsc

_NUM_CORES = 2
_NUM_SUBCORES = 16
_NUM_WORKERS = _NUM_CORES * _NUM_SUBCORES
_CHUNK = 1280


@functools.cache
def _make_gather(num_rows: int, dim: int, num_idx: int):
    assert num_idx % (_NUM_WORKERS * 2 * _CHUNK) == 0
    per_worker = num_idx // _NUM_WORKERS
    n_chunks = per_worker // _CHUNK
    mesh = plsc.VectorSubcoreMesh(
        core_axis_name="c",
        subcore_axis_name="s",
        num_cores=_NUM_CORES,
        num_subcores=_NUM_SUBCORES,
    )

    @functools.partial(
        pl.kernel,
        out_type=jax.ShapeDtypeStruct((num_idx, dim), jnp.float32),
        mesh=mesh,
        scratch_types=[
            pltpu.VMEM((n_chunks, _CHUNK), jnp.int32),
            pltpu.VMEM((2, _CHUNK, dim), jnp.float32),
            pltpu.SemaphoreType.DMA,
            pltpu.SemaphoreType.DMA,
        ],
    )
    def gather_kernel(idx_hbm, table_hbm, out_hbm, idx_v, rows_v, gsem, osem):
        wid = lax.axis_index("s") * _NUM_CORES + lax.axis_index("c")
        base = wid * per_worker
        pltpu.sync_copy(idx_hbm.at[pl.ds(wid * n_chunks, n_chunks)], idx_v)

        def start_gather(j, slot):
            pltpu.async_copy(table_hbm.at[idx_v.at[j]], rows_v.at[slot], gsem)

        def wait_gather(j, slot):
            pltpu.make_async_copy(
                table_hbm.at[idx_v.at[j]], rows_v.at[slot], gsem
            ).wait()

        def start_write(j, slot):
            off = pl.multiple_of(base + j * _CHUNK, _CHUNK)
            pltpu.async_copy(
                rows_v.at[slot], out_hbm.at[pl.ds(off, _CHUNK)], osem
            )

        def wait_write(j, slot):
            off = pl.multiple_of(base + j * _CHUNK, _CHUNK)
            pltpu.make_async_copy(
                rows_v.at[slot], out_hbm.at[pl.ds(off, _CHUNK)], osem
            ).wait()

        start_gather(0, 0)

        @pl.loop(0, n_chunks, step=2)
        def _chunk_loop(j0):
            for b in range(2):
                j = j0 + b
                wait_gather(j, b)

                @pl.when(j >= 1)
                def _():
                    wait_write(j - 1, 1 - b)

                @pl.when(j + 1 < n_chunks)
                def _():
                    start_gather(j + 1, 1 - b)

                start_write(j, b)

        wait_write(n_chunks - 1, (n_chunks - 1) % 2)

    return gather_kernel


def kernel(token_ids, weight):
    b, s = token_ids.shape
    num_rows, dim = weight.shape
    num_idx = b * s
    idx = token_ids.reshape(num_idx // _CHUNK, _CHUNK).astype(jnp.int32)
    out = _make_gather(num_rows, dim, num_idx)(idx, weight)
    return out.reshape(b, s, dim)


# speedup vs baseline: 1.1087x; 1.1087x over previous
import functools

import jax
import jax.numpy as jnp
from jax import lax
from jax.experimental import pallas as pl
from jax.experimental.pallas import tpu as pltpu
from jax.experimental.pallas import tpu_sc as plsc

_NUM_CORES = 2
_NUM_SUBCORES = 16
_NUM_WORKERS = _NUM_CORES * _NUM_SUBCORES
_CHUNK = 800


@functools.cache
def _make_gather(num_rows: int, dim: int, num_idx: int):
    assert num_idx % (_NUM_WORKERS * 2 * _CHUNK) == 0
    per_worker = num_idx // _NUM_WORKERS
    n_chunks = per_worker // _CHUNK
    mesh = plsc.VectorSubcoreMesh(
        core_axis_name="c",
        subcore_axis_name="s",
        num_cores=_NUM_CORES,
        num_subcores=_NUM_SUBCORES,
    )

    @functools.partial(
        pl.kernel,
        out_type=jax.ShapeDtypeStruct((num_idx, dim), jnp.float32),
        mesh=mesh,
        compiler_params=pltpu.CompilerParams(use_tc_tiling_on_sc=False),
        scratch_types=[
            pltpu.VMEM((n_chunks, _CHUNK), jnp.int32),
            pltpu.VMEM((2, _CHUNK, dim), jnp.float32),
            pltpu.SemaphoreType.DMA,
            pltpu.SemaphoreType.DMA,
        ],
    )
    def gather_kernel(idx_hbm, table_hbm, out_hbm, idx_v, rows_v, gsem, osem):
        wid = lax.axis_index("s") * _NUM_CORES + lax.axis_index("c")
        base = wid * per_worker
        pltpu.sync_copy(idx_hbm.at[pl.ds(wid * n_chunks, n_chunks)], idx_v)

        def start_gather(j, slot):
            pltpu.async_copy(table_hbm.at[idx_v.at[j]], rows_v.at[slot], gsem)

        def wait_gather(j, slot):
            pltpu.make_async_copy(
                table_hbm.at[idx_v.at[j]], rows_v.at[slot], gsem
            ).wait()

        def start_write(j, slot):
            off = pl.multiple_of(base + j * _CHUNK, _CHUNK)
            pltpu.async_copy(
                rows_v.at[slot], out_hbm.at[pl.ds(off, _CHUNK)], osem
            )

        def wait_write(j, slot):
            off = pl.multiple_of(base + j * _CHUNK, _CHUNK)
            pltpu.make_async_copy(
                rows_v.at[slot], out_hbm.at[pl.ds(off, _CHUNK)], osem
            ).wait()

        start_gather(0, 0)

        @pl.loop(0, n_chunks, step=2)
        def _chunk_loop(j0):
            for b in range(2):
                j = j0 + b
                wait_gather(j, b)

                @pl.when(j >= 1)
                def _():
                    wait_write(j - 1, 1 - b)

                @pl.when(j + 1 < n_chunks)
                def _():
                    start_gather(j + 1, 1 - b)

                start_write(j, b)

        wait_write(n_chunks - 1, (n_chunks - 1) % 2)

    return gather_kernel


def kernel(token_ids, weight):
    b, s = token_ids.shape
    num_rows, dim = weight.shape
    num_idx = b * s
    idx = token_ids.reshape(num_idx // _CHUNK, _CHUNK).astype(jnp.int32)
    out = _make_gather(num_rows, dim, num_idx)(idx, weight)
    return out.reshape(b, s, dim)


# speedup vs baseline: 1.4658x; 1.3220x over previous
import functools

import jax
import jax.numpy as jnp
from jax import lax
from jax.experimental import pallas as pl
from jax.experimental.pallas import tpu as pltpu
from jax.experimental.pallas import tpu_sc as plsc

_NUM_CORES = 2
_NUM_SUBCORES = 16
_NUM_WORKERS = _NUM_CORES * _NUM_SUBCORES
_LANES = 16


@functools.cache
def _make_gather(num_rows: int, dim: int, seq: int, batch: int):
    bb = batch // _NUM_WORKERS
    assert bb % 128 == 0 and seq % 2 == 0
    mesh = plsc.VectorSubcoreMesh(
        core_axis_name="c",
        subcore_axis_name="s",
        num_cores=_NUM_CORES,
        num_subcores=_NUM_SUBCORES,
    )

    @functools.partial(
        pl.kernel,
        out_type=jax.ShapeDtypeStruct((seq, dim, batch), jnp.float32),
        mesh=mesh,
        compiler_params=pltpu.CompilerParams(
            use_tc_tiling_on_sc=False, needs_layout_passes=False
        ),
        scratch_types=[
            pltpu.VMEM((seq, bb), jnp.int32),
            pltpu.VMEM((2, bb, dim), jnp.float32),
            pltpu.VMEM((2, dim, bb), jnp.float32),
            pltpu.SemaphoreType.DMA,
            pltpu.SemaphoreType.DMA,
        ],
    )
    def gather_kernel(tokt_hbm, w_hbm, out_hbm, idx_v, rows_v, trans_v,
                      gsem, osem):
        wid = lax.axis_index("s") * _NUM_CORES + lax.axis_index("c")
        b0 = pl.multiple_of(wid * bb, bb)
        pltpu.sync_copy(tokt_hbm.at[:, pl.ds(b0, bb)], idx_v)

        def start_gather(s, slot):
            pltpu.async_copy(w_hbm.at[idx_v.at[s]], rows_v.at[slot], gsem)

        def wait_gather(s, slot):
            pltpu.make_async_copy(
                w_hbm.at[idx_v.at[s]], rows_v.at[slot], gsem
            ).wait()

        def start_write(s, slot):
            pltpu.async_copy(
                trans_v.at[slot], out_hbm.at[s, :, pl.ds(b0, bb)], osem
            )

        def wait_write(s, slot):
            pltpu.make_async_copy(
                trans_v.at[slot], out_hbm.at[s, :, pl.ds(b0, bb)], osem
            ).wait()

        def transpose_chunk(slot):
            src = rows_v.at[slot]
            dst = trans_v.at[slot]

            @pl.loop(0, bb, step=_LANES)
            def _t(k0):
                ridx = k0 + lax.iota(jnp.int32, _LANES)
                for d in range(dim):
                    cidx = jnp.full((_LANES,), d, jnp.int32)
                    dst[d, pl.ds(k0, _LANES)] = plsc.load_gather(
                        src, [ridx, cidx]
                    )

        start_gather(0, 0)

        @pl.loop(0, seq, step=2)
        def _chunk_loop(s0):
            for b in range(2):
                s = s0 + b
                wait_gather(s, b)

                @pl.when(s + 1 < seq)
                def _():
                    start_gather(s + 1, 1 - b)

                @pl.when(s >= 1)
                def _():
                    wait_write(s - 1, 1 - b)

                transpose_chunk(b)
                start_write(s, b)

        wait_write(seq - 1, (seq - 1) % 2)

    return gather_kernel


def kernel(token_ids, weight):
    b, s = token_ids.shape
    num_rows, dim = weight.shape
    tokt = token_ids.T.astype(jnp.int32)
    out3 = _make_gather(num_rows, dim, s, b)(tokt, weight)
    return jnp.transpose(out3, (2, 0, 1))


# speedup vs baseline: 1.7629x; 1.2027x over previous
import functools

import jax
import jax.numpy as jnp
from jax import lax
from jax.experimental import pallas as pl
from jax.experimental.pallas import tpu as pltpu
from jax.experimental.pallas import tpu_sc as plsc

_NUM_CORES = 2
_NUM_SUBCORES = 16
_NUM_WORKERS = _NUM_CORES * _NUM_SUBCORES
_LANES = 16


@functools.cache
def _make_gather(num_rows: int, dim: int, seq: int, batch: int):
    bb = batch // _NUM_WORKERS
    assert bb % 128 == 0 and seq % 2 == 0
    mesh = plsc.VectorSubcoreMesh(
        core_axis_name="c",
        subcore_axis_name="s",
        num_cores=_NUM_CORES,
        num_subcores=_NUM_SUBCORES,
    )

    @functools.partial(
        pl.kernel,
        out_type=jax.ShapeDtypeStruct((seq, dim, batch), jnp.float32),
        mesh=mesh,
        compiler_params=pltpu.CompilerParams(
            use_tc_tiling_on_sc=False, needs_layout_passes=False
        ),
        scratch_types=[
            pltpu.VMEM((seq, bb), jnp.int32),
            pltpu.VMEM((2, bb, dim), jnp.float32),
            pltpu.VMEM((2, dim, bb), jnp.float32),
            pltpu.SemaphoreType.DMA,
            pltpu.SemaphoreType.DMA,
        ],
    )
    def gather_kernel(tokt_hbm, w_hbm, out_hbm, idx_v, rows_v, trans_v,
                      gsem, osem):
        wid = lax.axis_index("s") * _NUM_CORES + lax.axis_index("c")
        b0 = pl.multiple_of(wid * bb, bb)
        pltpu.sync_copy(tokt_hbm.at[:, pl.ds(b0, bb)], idx_v)

        def start_gather(s, slot):
            pltpu.async_copy(w_hbm.at[idx_v.at[s]], rows_v.at[slot], gsem)

        def wait_gather(s, slot):
            pltpu.make_async_copy(
                w_hbm.at[idx_v.at[s]], rows_v.at[slot], gsem
            ).wait()

        def start_write(s, slot):
            pltpu.async_copy(
                trans_v.at[slot], out_hbm.at[s, :, pl.ds(b0, bb)], osem
            )

        def wait_write(s, slot):
            pltpu.make_async_copy(
                trans_v.at[slot], out_hbm.at[s, :, pl.ds(b0, bb)], osem
            ).wait()

        def transpose_chunk(slot):
            src = rows_v.at[slot]
            dst = trans_v.at[slot]

            @plsc.parallel_loop(0, bb, step=_LANES, unroll=4)
            def _t(k0):
                ridx = k0 + lax.iota(jnp.int32, _LANES)
                for d in range(dim):
                    cidx = jnp.full((_LANES,), d, jnp.int32)
                    dst[d, pl.ds(k0, _LANES)] = plsc.load_gather(
                        src, [ridx, cidx]
                    )

        start_gather(0, 0)

        @pl.loop(0, seq, step=2)
        def _chunk_loop(s0):
            for b in range(2):
                s = s0 + b
                wait_gather(s, b)

                @pl.when(s + 1 < seq)
                def _():
                    start_gather(s + 1, 1 - b)

                @pl.when(s >= 1)
                def _():
                    wait_write(s - 1, 1 - b)

                transpose_chunk(b)
                start_write(s, b)

        wait_write(seq - 1, (seq - 1) % 2)

    return gather_kernel


def kernel(token_ids, weight):
    b, s = token_ids.shape
    num_rows, dim = weight.shape
    tokt = token_ids.T.astype(jnp.int32)
    out3 = _make_gather(num_rows, dim, s, b)(tokt, weight)
    return jnp.transpose(out3, (2, 0, 1))


# speedup vs baseline: 2.6545x; 1.5058x over previous
import functools

import jax
import jax.numpy as jnp
from jax import lax
from jax.experimental import pallas as pl
from jax.experimental.pallas import tpu as pltpu
from jax.experimental.pallas import tpu_sc as plsc

_NUM_CORES = 2
_NUM_SUBCORES = 16
_NUM_WORKERS = _NUM_CORES * _NUM_SUBCORES
_LANES = 16


@functools.cache
def _make_gather(num_rows: int, dim: int, seq: int, batch: int):
    bb = batch // _NUM_WORKERS
    assert bb % 128 == 0 and seq % 2 == 0
    mesh = plsc.VectorSubcoreMesh(
        core_axis_name="c",
        subcore_axis_name="s",
        num_cores=_NUM_CORES,
        num_subcores=_NUM_SUBCORES,
    )

    @functools.partial(
        pl.kernel,
        out_type=jax.ShapeDtypeStruct((seq, dim, batch), jnp.float32),
        mesh=mesh,
        compiler_params=pltpu.CompilerParams(
            use_tc_tiling_on_sc=False, needs_layout_passes=False
        ),
        scratch_types=[
            pltpu.VMEM((seq, bb), jnp.int32),
            pltpu.VMEM((2, bb, dim), jnp.float32),
            pltpu.VMEM((2, dim, bb + 1), jnp.float32),
            pltpu.SemaphoreType.DMA,
            pltpu.SemaphoreType.DMA,
        ],
    )
    def gather_kernel(tokt_hbm, w_hbm, out_hbm, idx_v, rows_v, trans_v,
                      gsem, osem):
        wid = lax.axis_index("s") * _NUM_CORES + lax.axis_index("c")
        b0 = pl.multiple_of(wid * bb, bb)
        pltpu.sync_copy(tokt_hbm.at[:, pl.ds(b0, bb)], idx_v)

        def start_gather(s, slot):
            pltpu.async_copy(w_hbm.at[idx_v.at[s]], rows_v.at[slot], gsem)

        def wait_gather(s, slot):
            pltpu.make_async_copy(
                w_hbm.at[idx_v.at[s]], rows_v.at[slot], gsem
            ).wait()

        def start_write(s, slot):
            pltpu.async_copy(
                trans_v.at[slot, :, pl.ds(0, bb)],
                out_hbm.at[s, :, pl.ds(b0, bb)],
                osem,
            )

        def wait_write(s, slot):
            pltpu.make_async_copy(
                trans_v.at[slot, :, pl.ds(0, bb)],
                out_hbm.at[s, :, pl.ds(b0, bb)],
                osem,
            ).wait()

        def transpose_chunk(slot):
            src = rows_v.at[slot]
            dst = trans_v.at[slot]

            @plsc.parallel_loop(0, bb, step=1, unroll=8)
            def _t(r):
                kidx = jnp.full((_LANES,), r, jnp.int32)
                for h in range(dim // _LANES):
                    cidx = h * _LANES + lax.iota(jnp.int32, _LANES)
                    vals = src[r, pl.ds(h * _LANES, _LANES)]
                    plsc.store_scatter(dst, [cidx, kidx], vals)

        start_gather(0, 0)

        @pl.loop(0, seq, step=2)
        def _chunk_loop(s0):
            for b in range(2):
                s = s0 + b
                wait_gather(s, b)

                @pl.when(s + 1 < seq)
                def _():
                    start_gather(s + 1, 1 - b)

                @pl.when(s >= 1)
                def _():
                    wait_write(s - 1, 1 - b)

                transpose_chunk(b)
                start_write(s, b)

        wait_write(seq - 1, (seq - 1) % 2)

    return gather_kernel


def kernel(token_ids, weight):
    b, s = token_ids.shape
    num_rows, dim = weight.shape
    tokt = token_ids.T.astype(jnp.int32)
    out3 = _make_gather(num_rows, dim, s, b)(tokt, weight)
    return jnp.transpose(out3, (2, 0, 1))
